# baseline (device time: 35262 ns/iter reference)
import jax
import jax.numpy as jnp
from jax import lax
from jax.experimental import pallas as pl
from jax.experimental.pallas import tpu as pltpu

N_DEV = 4
B, Sq, Hq, Dh = 2, 128, 4, 64
BLK = 64


def kernel(x, Wq, K_ext, V_ext, Wo):
    skv_loc = K_ext.shape[1]
    d_model = x.shape[-1]

    def body(x_ref, wq_ref, k_ref, v_ref, wo_ref, out_ref,
             kv_buf, send_sems, recv_sems, copy_sems, exit_sem):
        my = lax.axis_index("i")

        barrier = pltpu.get_barrier_semaphore()

        @pl.when(my == 0)
        def _():
            for d in range(1, N_DEV):
                pl.semaphore_signal(
                    barrier, inc=1, device_id=(d,),
                    device_id_type=pl.DeviceIdType.MESH)
            pl.semaphore_wait(barrier, N_DEV - 1)

        @pl.when(my != 0)
        def _():
            pl.semaphore_signal(
                barrier, inc=1, device_id=(0,),
                device_id_type=pl.DeviceIdType.MESH)
            pl.semaphore_wait(barrier, 1)

        sends = []
        for idx, d in enumerate(range(1, N_DEV)):
            for s, src in enumerate((k_ref, v_ref)):
                sends.append(pltpu.make_async_remote_copy(
                    src_ref=src,
                    dst_ref=kv_buf.at[s],
                    send_sem=send_sems.at[2 * idx + s],
                    recv_sem=recv_sems.at[s],
                    device_id=(d,),
                    device_id_type=pl.DeviceIdType.MESH,
                ))
        copy_k = pltpu.make_async_copy(k_ref, kv_buf.at[0], copy_sems.at[0])
        copy_v = pltpu.make_async_copy(v_ref, kv_buf.at[1], copy_sems.at[1])
        recv_k = pltpu.make_async_remote_copy(
            src_ref=kv_buf.at[0], dst_ref=kv_buf.at[0],
            send_sem=send_sems.at[0], recv_sem=recv_sems.at[0],
            device_id=(0,), device_id_type=pl.DeviceIdType.MESH)
        recv_v = pltpu.make_async_remote_copy(
            src_ref=kv_buf.at[1], dst_ref=kv_buf.at[1],
            send_sem=send_sems.at[1], recv_sem=recv_sems.at[1],
            device_id=(0,), device_id_type=pl.DeviceIdType.MESH)

        @pl.when(my == 0)
        def _():
            for r in sends:
                r.start()
            copy_k.start()
            copy_v.start()

        wq = wq_ref[...]
        q_alls = [
            jnp.dot(x_ref[b], wq, preferred_element_type=jnp.float32) * 0.125
            for b in range(B)
        ]

        @pl.when(my == 0)
        def _():
            copy_k.wait()
            copy_v.wait()

        @pl.when(my != 0)
        def _():
            recv_k.wait_recv()
            recv_v.wait_recv()

        wo = wo_ref[...]
        for b in range(B):
            q_all = q_alls[b]
            ctx_heads = []
            for h in range(Hq):
                k_bh = kv_buf[0, b, :, h, :]
                v_bh = kv_buf[1, b, :, h, :]
                q0 = q_all[0:BLK, h * Dh:(h + 1) * Dh]
                q1 = q_all[BLK:Sq, h * Dh:(h + 1) * Dh]
                s0 = lax.dot_general(
                    q0, k_bh[0:BLK, :], (((1,), (1,)), ((), ())),
                    preferred_element_type=jnp.float32)
                s1 = lax.dot_general(
                    q1, k_bh, (((1,), (1,)), ((), ())),
                    preferred_element_type=jnp.float32)
                w0 = jnp.exp(s0 - jnp.max(s0, axis=1, keepdims=True))
                w0 = w0 / jnp.sum(w0, axis=1, keepdims=True)
                w1 = jnp.exp(s1 - jnp.max(s1, axis=1, keepdims=True))
                w1 = w1 / jnp.sum(w1, axis=1, keepdims=True)
                c0 = jnp.dot(w0, v_bh[0:BLK, :],
                             preferred_element_type=jnp.float32)
                c1 = jnp.dot(w1, v_bh,
                             preferred_element_type=jnp.float32)
                ctx_heads.append(jnp.concatenate([c0, c1], axis=0))
            ctx = jnp.concatenate(ctx_heads, axis=1)
            out_ref[b] = jnp.dot(ctx, wo,
                                 preferred_element_type=jnp.float32)

        @pl.when(my == 0)
        def _():
            for r in sends:
                r.wait_send()
            pl.semaphore_wait(exit_sem, N_DEV - 1)

        @pl.when(my != 0)
        def _():
            pl.semaphore_signal(
                exit_sem, inc=1, device_id=(0,),
                device_id_type=pl.DeviceIdType.MESH)

    return pl.pallas_call(
        body,
        out_shape=jax.ShapeDtypeStruct((B, Sq, d_model), jnp.float32),
        in_specs=[pl.BlockSpec(memory_space=pltpu.VMEM)] * 5,
        out_specs=pl.BlockSpec(memory_space=pltpu.VMEM),
        scratch_shapes=[
            pltpu.VMEM((2, B, skv_loc, Hq, Dh), jnp.float32),
            pltpu.SemaphoreType.DMA((2 * (N_DEV - 1),)),
            pltpu.SemaphoreType.DMA((2,)),
            pltpu.SemaphoreType.DMA((2,)),
            pltpu.SemaphoreType.REGULAR,
        ],
        compiler_params=pltpu.CompilerParams(collective_id=0),
    )(x, Wq, K_ext, V_ext, Wo)


# device time: 7028 ns/iter; 5.0174x vs baseline; 5.0174x over previous
import jax
import jax.numpy as jnp
from jax import lax
from jax.experimental import pallas as pl
from jax.experimental.pallas import tpu as pltpu

N_DEV = 4
B, Sq, Hq, Dh = 2, 128, 4, 64
BLK = 64


def kernel(x, Wq, K_ext, V_ext, Wo):
    skv_loc = K_ext.shape[1]
    d_model = x.shape[-1]

    def body(x_ref, wq_ref, k_ref, v_ref, wo_ref, out_ref):
        wq = wq_ref[...]
        q_alls = [
            jnp.dot(x_ref[b], wq, preferred_element_type=jnp.float32) * 0.125
            for b in range(B)
        ]
        wo = wo_ref[...]
        for b in range(B):
            q_all = q_alls[b]
            ctx_heads = []
            for h in range(Hq):
                k_bh = k_ref[b, :, h, :]
                v_bh = v_ref[b, :, h, :]
                q0 = q_all[0:BLK, h * Dh:(h + 1) * Dh]
                q1 = q_all[BLK:Sq, h * Dh:(h + 1) * Dh]
                s0 = lax.dot_general(
                    q0, k_bh[0:BLK, :], (((1,), (1,)), ((), ())),
                    preferred_element_type=jnp.float32)
                s1 = lax.dot_general(
                    q1, k_bh, (((1,), (1,)), ((), ())),
                    preferred_element_type=jnp.float32)
                w0 = jnp.exp(s0 - jnp.max(s0, axis=1, keepdims=True))
                w0 = w0 / jnp.sum(w0, axis=1, keepdims=True)
                w1 = jnp.exp(s1 - jnp.max(s1, axis=1, keepdims=True))
                w1 = w1 / jnp.sum(w1, axis=1, keepdims=True)
                c0 = jnp.dot(w0, v_bh[0:BLK, :],
                             preferred_element_type=jnp.float32)
                c1 = jnp.dot(w1, v_bh,
                             preferred_element_type=jnp.float32)
                ctx_heads.append(jnp.concatenate([c0, c1], axis=0))
            ctx = jnp.concatenate(ctx_heads, axis=1)
            out_ref[b] = jnp.dot(ctx, wo,
                                 preferred_element_type=jnp.float32)

    return pl.pallas_call(
        body,
        out_shape=jax.ShapeDtypeStruct((B, Sq, d_model), jnp.float32),
        in_specs=[pl.BlockSpec(memory_space=pltpu.VMEM)] * 5,
        out_specs=pl.BlockSpec(memory_space=pltpu.VMEM),
    )(x, Wq, K_ext, V_ext, Wo)
